# baseline (speedup 1.0000x reference)
import functools

import jax
import jax.numpy as jnp
from jax import lax
from jax.experimental import pallas as pl
from jax.experimental.pallas import tpu as pltpu
from jax.experimental.pallas import tpu_sc as plsc

N = 10000
E = 320000
D = 128
R = 64
AGG = 32
OUT = 10

EB = 128
NUM_BATCHES = E // EB
NW = 32
BATCHES_PER_TILE = -(-NUM_BATCHES // NW)
NPAD = 10240
ROWS_PER_TILE = NPAD // 16
CHUNK = 128


def _sc_edge_accumulate(x, packed, w_conv):
    mesh = plsc.VectorSubcoreMesh(core_axis_name="c", subcore_axis_name="s")

    @functools.partial(
        pl.kernel,
        mesh=mesh,
        compiler_params=pltpu.CompilerParams(needs_layout_passes=False),
        out_type=jax.ShapeDtypeStruct((2, NPAD, D), jnp.float32),
        scratch_types=[
            pltpu.VMEM((8, EB), jnp.int32),
            pltpu.VMEM((8, EB), jnp.int32),
            pltpu.VMEM((EB, D), jnp.float32),
            pltpu.VMEM((EB, D), jnp.float32),
            pltpu.VMEM((128,), jnp.float32),
            pltpu.VMEM((128,), jnp.float32),
            pltpu.VMEM_SHARED((NPAD, D), jnp.float32),
            pltpu.SemaphoreType.DMA,
            pltpu.SemaphoreType.DMA,
        ],
    )
    def k(x_hbm, packed_hbm, wconv_hbm, out_hbm,
          idx0, idx1, rows0, rows1, wconv_v, wg_v, h_sh, sem0, sem1):
        c = lax.axis_index("c")
        s = lax.axis_index("s")
        wid = s * 2 + c

        pltpu.sync_copy(wconv_hbm, wconv_v.at[pl.ds(0, R)])

        z16 = jnp.zeros((16,), jnp.float32)

        def zrow(i, _):
            for jb in range(D // 16):
                rows0[i, pl.ds(jb * 16, 16)] = z16
            return 0

        lax.fori_loop(0, EB, zrow, 0)
        for t in range(ROWS_PER_TILE // CHUNK):
            pltpu.sync_copy(
                rows0.at[pl.ds(0, CHUNK)],
                h_sh.at[pl.ds(s * ROWS_PER_TILE + t * CHUNK, CHUNK)])
        plsc.subcore_barrier()

        def run_batch(b, idx_v, rows_v, sem, nidx_v, nrows_v, nsem):
            @pl.when(b < NUM_BATCHES)
            def _():
                pltpu.make_async_copy(x_hbm.at[idx_v.at[0]], rows_v, sem).wait()
                bn = b + NW

                @pl.when(bn < NUM_BATCHES)
                def _():
                    pltpu.sync_copy(packed_hbm.at[bn], nidx_v)
                    pltpu.async_copy(x_hbm.at[nidx_v.at[0]], nrows_v, nsem)

                for g in range(EB // 16):
                    r16 = idx_v[2, pl.ds(g * 16, 16)]
                    wg_v[pl.ds(g * 16, 16)] = plsc.load_gather(wconv_v, [r16])

                def scale_row(i, _):
                    i16 = jnp.zeros((16,), jnp.int32) + i
                    wb = plsc.load_gather(wg_v, [i16])
                    for jb in range(D // 16):
                        rows_v[i, pl.ds(jb * 16, 16)] = (
                            rows_v[i, pl.ds(jb * 16, 16)] * wb)
                    return 0

                lax.fori_loop(0, EB, scale_row, 0, unroll=4)
                pltpu.sync_copy(rows_v, h_sh.at[idx_v.at[1]], add=True)

        pltpu.sync_copy(packed_hbm.at[wid], idx0)
        pltpu.async_copy(x_hbm.at[idx0.at[0]], rows0, sem0)

        def batch_pair(t, _):
            b = wid + (2 * t) * NW
            run_batch(b, idx0, rows0, sem0, idx1, rows1, sem1)
            run_batch(b + NW, idx1, rows1, sem1, idx0, rows0, sem0)
            return 0

        lax.fori_loop(0, (BATCHES_PER_TILE + 1) // 2, batch_pair, 0)
        plsc.subcore_barrier()

        for t in range(ROWS_PER_TILE // CHUNK):
            r0 = s * ROWS_PER_TILE + t * CHUNK
            pltpu.sync_copy(h_sh.at[pl.ds(r0, CHUNK)],
                            rows0.at[pl.ds(0, CHUNK)])
            pltpu.sync_copy(rows0.at[pl.ds(0, CHUNK)],
                            out_hbm.at[c, pl.ds(r0, CHUNK)])

    return k(x, packed, w_conv)


NB = 1000
GRID = N // NB


def _tc_head(hp, nrA, nrB, bconv_row, w_agg, Wt, blin2):
    def body(hp_ref, nrA_ref, nrB_ref, bc_ref, wagg_ref, wt_ref, bl_ref,
             out_ref, s_ref):
        i = pl.program_id(0)

        @pl.when(i == 0)
        def _():
            s_ref[...] = jnp.zeros((AGG, D), jnp.float32)

        acc = hp_ref[0] + hp_ref[1]
        nrcol = nrA_ref[0]
        oh = (lax.broadcasted_iota(jnp.int32, (NB, R), 1) == nrcol
              ).astype(jnp.float32)
        bcol = jnp.sum(oh * bc_ref[...], axis=1, keepdims=True)
        h = jnp.tanh(acc + bcol)
        nrrow = nrB_ref[0]
        ohT = (lax.broadcasted_iota(jnp.int32, (R, NB), 0) == nrrow
               ).astype(jnp.float32)
        wn = jnp.dot(wagg_ref[...], ohT,
                     preferred_element_type=jnp.float32)
        s_ref[...] += jnp.dot(wn, h, preferred_element_type=jnp.float32)

        @pl.when(i == GRID - 1)
        def _():
            a = jnp.tanh(s_ref[...])
            acc10 = bl_ref[...]
            for k in range(AGG):
                acc10 = acc10 + jnp.dot(a[k:k + 1, :], wt_ref[k],
                                        preferred_element_type=jnp.float32)
            out_ref[...] = jnp.tanh(acc10)

    return pl.pallas_call(
        body,
        grid=(GRID,),
        in_specs=[
            pl.BlockSpec((2, NB, D), lambda i: (0, i, 0)),
            pl.BlockSpec((1, NB, 1), lambda i: (i, 0, 0)),
            pl.BlockSpec((1, 1, NB), lambda i: (i, 0, 0)),
            pl.BlockSpec((1, R), lambda i: (0, 0)),
            pl.BlockSpec((AGG, R), lambda i: (0, 0)),
            pl.BlockSpec((AGG, D, OUT), lambda i: (0, 0, 0)),
            pl.BlockSpec((1, OUT), lambda i: (0, 0)),
        ],
        out_specs=pl.BlockSpec((1, OUT), lambda i: (0, 0)),
        out_shape=jax.ShapeDtypeStruct((1, OUT), jnp.float32),
        scratch_shapes=[pltpu.VMEM((AGG, D), jnp.float32)],
    )(hp, nrA, nrB, bconv_row, w_agg, Wt, blin2)


def kernel(x, pos, edge_rule, node_rule, w_conv, b_conv, w_agg, W_lin, b_lin):
    src = pos[0].reshape(NUM_BATCHES, 1, EB)
    dst = pos[1].reshape(NUM_BATCHES, 1, EB)
    erule = edge_rule.reshape(NUM_BATCHES, 1, EB)
    pad = jnp.zeros((NUM_BATCHES, 5, EB), jnp.int32)
    packed = jnp.concatenate([src, dst, erule, pad], axis=1)
    hp = _sc_edge_accumulate(x, packed, w_conv)[:, :N, :]
    nrA = node_rule.reshape(GRID, NB, 1)
    nrB = node_rule.reshape(GRID, 1, NB)
    Wt = W_lin.reshape(OUT, AGG, D).transpose(1, 2, 0)
    out2 = _tc_head(hp, nrA, nrB, b_conv.reshape(1, R), w_agg, Wt,
                    b_lin.reshape(1, OUT))
    return out2.reshape(-1)


# speedup vs baseline: 1.0928x; 1.0928x over previous
import functools

import jax
import jax.numpy as jnp
from jax import lax
from jax.experimental import pallas as pl
from jax.experimental.pallas import tpu as pltpu
from jax.experimental.pallas import tpu_sc as plsc

N = 10000
E = 320000
D = 128
R = 64
AGG = 32
OUT = 10

EB = 128
NUM_BATCHES = E // EB
NW = 32
BATCHES_PER_TILE = -(-NUM_BATCHES // NW)
NPAD = 10240
ROWS_PER_TILE = NPAD // 16
CHUNK = 128


PHASES = 2
PHASE_BATCHES = 40
IDX_ROWS = 3 * PHASES * PHASE_BATCHES
IDX_ROWS_VMEM = 3 * PHASE_BATCHES


def _sc_edge_accumulate(x, perm, w_conv):
    mesh = plsc.VectorSubcoreMesh(core_axis_name="c", subcore_axis_name="s")

    @functools.partial(
        pl.kernel,
        mesh=mesh,
        compiler_params=pltpu.CompilerParams(needs_layout_passes=False),
        out_type=jax.ShapeDtypeStruct((2, NPAD, D), jnp.float32),
        scratch_types=[
            pltpu.VMEM((IDX_ROWS_VMEM, EB), jnp.int32),
            pltpu.VMEM((EB, D), jnp.float32),
            pltpu.VMEM((EB, D), jnp.float32),
            pltpu.VMEM((128,), jnp.float32),
            pltpu.VMEM((128,), jnp.float32),
            pltpu.VMEM_SHARED((NPAD, D), jnp.float32),
            pltpu.SemaphoreType.DMA,
            pltpu.SemaphoreType.DMA,
        ],
    )
    def k(x_hbm, perm_hbm, wconv_hbm, out_hbm,
          idx_all, rows0, rows1, wconv_v, wg_v, h_sh, sem0, sem1):
        c = lax.axis_index("c")
        s = lax.axis_index("s")
        wid = s * 2 + c

        pltpu.sync_copy(wconv_hbm, wconv_v.at[pl.ds(0, R)])
        pltpu.sync_copy(perm_hbm.at[wid, pl.ds(0, IDX_ROWS_VMEM)], idx_all)

        z16 = jnp.zeros((16,), jnp.float32)

        def zrow(i, _):
            for jb in range(D // 16):
                rows0[i, pl.ds(jb * 16, 16)] = z16
            return 0

        lax.fori_loop(0, EB, zrow, 0)
        for t in range(ROWS_PER_TILE // CHUNK):
            pltpu.sync_copy(
                rows0.at[pl.ds(0, CHUNK)],
                h_sh.at[pl.ds(s * ROWS_PER_TILE + t * CHUNK, CHUNK)])
        plsc.subcore_barrier()

        def run_batch(kk, kbase, rows_v, sem, nrows_v, nsem):
            b = wid + (kbase + kk) * NW

            @pl.when(b < NUM_BATCHES)
            def _():
                r0 = 3 * kk
                pltpu.make_async_copy(
                    x_hbm.at[idx_all.at[r0]], rows_v, sem).wait()
                bn = b + NW

                @pl.when(jnp.logical_and(bn < NUM_BATCHES,
                                         kk < PHASE_BATCHES - 1))
                def _():
                    pltpu.async_copy(
                        x_hbm.at[idx_all.at[r0 + 3]], nrows_v, nsem)

                for g in range(EB // 16):
                    r16 = idx_all[r0 + 2, pl.ds(g * 16, 16)]
                    wg_v[pl.ds(g * 16, 16)] = plsc.load_gather(wconv_v, [r16])

                def scale_row(i, _):
                    i16 = jnp.zeros((16,), jnp.int32) + i
                    wb = plsc.load_gather(wg_v, [i16])
                    for jb in range(D // 16):
                        rows_v[i, pl.ds(jb * 16, 16)] = (
                            rows_v[i, pl.ds(jb * 16, 16)] * wb)
                    return 0

                lax.fori_loop(0, EB, scale_row, 0, unroll=4)
                pltpu.sync_copy(rows_v, h_sh.at[idx_all.at[r0 + 1]], add=True)

        for phase in range(PHASES):
            kbase = phase * PHASE_BATCHES
            pltpu.sync_copy(
                perm_hbm.at[wid, pl.ds(3 * kbase, 3 * PHASE_BATCHES)],
                idx_all)

            @pl.when(wid + kbase * NW < NUM_BATCHES)
            def _():
                pltpu.async_copy(x_hbm.at[idx_all.at[0]], rows0, sem0)

            def batch_pair(t, _, kbase=kbase):
                kk = 2 * t
                run_batch(kk, kbase, rows0, sem0, rows1, sem1)
                run_batch(kk + 1, kbase, rows1, sem1, rows0, sem0)
                return 0

            lax.fori_loop(0, PHASE_BATCHES // 2, batch_pair, 0)
        plsc.subcore_barrier()

        for t in range(ROWS_PER_TILE // CHUNK):
            r0 = s * ROWS_PER_TILE + t * CHUNK
            pltpu.sync_copy(h_sh.at[pl.ds(r0, CHUNK)],
                            rows0.at[pl.ds(0, CHUNK)])
            pltpu.sync_copy(rows0.at[pl.ds(0, CHUNK)],
                            out_hbm.at[c, pl.ds(r0, CHUNK)])

    return k(x, perm, w_conv)


NB = 1000
GRID = N // NB


def _tc_head(hp, nrA, nrB, bconv_row, w_agg, Wt, blin2):
    def body(hp_ref, nrA_ref, nrB_ref, bc_ref, wagg_ref, wt_ref, bl_ref,
             out_ref, s_ref):
        i = pl.program_id(0)

        @pl.when(i == 0)
        def _():
            s_ref[...] = jnp.zeros((AGG, D), jnp.float32)

        acc = hp_ref[0] + hp_ref[1]
        nrcol = nrA_ref[0]
        oh = (lax.broadcasted_iota(jnp.int32, (NB, R), 1) == nrcol
              ).astype(jnp.float32)
        bcol = jnp.sum(oh * bc_ref[...], axis=1, keepdims=True)
        h = jnp.tanh(acc + bcol)
        nrrow = nrB_ref[0]
        ohT = (lax.broadcasted_iota(jnp.int32, (R, NB), 0) == nrrow
               ).astype(jnp.float32)
        wn = jnp.dot(wagg_ref[...], ohT,
                     preferred_element_type=jnp.float32)
        s_ref[...] += jnp.dot(wn, h, preferred_element_type=jnp.float32)

        @pl.when(i == GRID - 1)
        def _():
            a = jnp.tanh(s_ref[...])
            acc10 = bl_ref[...]
            for k in range(AGG):
                acc10 = acc10 + jnp.dot(a[k:k + 1, :], wt_ref[k],
                                        preferred_element_type=jnp.float32)
            out_ref[...] = jnp.tanh(acc10)

    return pl.pallas_call(
        body,
        grid=(GRID,),
        in_specs=[
            pl.BlockSpec((2, NB, D), lambda i: (0, i, 0)),
            pl.BlockSpec((1, NB, 1), lambda i: (i, 0, 0)),
            pl.BlockSpec((1, 1, NB), lambda i: (i, 0, 0)),
            pl.BlockSpec((1, R), lambda i: (0, 0)),
            pl.BlockSpec((AGG, R), lambda i: (0, 0)),
            pl.BlockSpec((AGG, D, OUT), lambda i: (0, 0, 0)),
            pl.BlockSpec((1, OUT), lambda i: (0, 0)),
        ],
        out_specs=pl.BlockSpec((1, OUT), lambda i: (0, 0)),
        out_shape=jax.ShapeDtypeStruct((1, OUT), jnp.float32),
        scratch_shapes=[pltpu.VMEM((AGG, D), jnp.float32)],
    )(hp, nrA, nrB, bconv_row, w_agg, Wt, blin2)


def kernel(x, pos, edge_rule, node_rule, w_conv, b_conv, w_agg, W_lin, b_lin):
    src = pos[0].reshape(NUM_BATCHES, 1, EB)
    dst = pos[1].reshape(NUM_BATCHES, 1, EB)
    erule = edge_rule.reshape(NUM_BATCHES, 1, EB)
    packed = jnp.concatenate([src, dst, erule], axis=1)
    bidx = (jnp.arange(NW)[:, None] +
            jnp.arange(PHASES * PHASE_BATCHES)[None, :] * NW)
    bidx = jnp.where(bidx < NUM_BATCHES, bidx, 0)
    perm = packed[bidx].reshape(NW, IDX_ROWS, EB)
    hp = _sc_edge_accumulate(x, perm, w_conv)[:, :N, :]
    nrA = node_rule.reshape(GRID, NB, 1)
    nrB = node_rule.reshape(GRID, 1, NB)
    Wt = W_lin.reshape(OUT, AGG, D).transpose(1, 2, 0)
    out2 = _tc_head(hp, nrA, nrB, b_conv.reshape(1, R), w_agg, Wt,
                    b_lin.reshape(1, OUT))
    return out2.reshape(-1)


# speedup vs baseline: 1.0936x; 1.0007x over previous
import functools

import jax
import jax.numpy as jnp
from jax import lax
from jax.experimental import pallas as pl
from jax.experimental.pallas import tpu as pltpu
from jax.experimental.pallas import tpu_sc as plsc

N = 10000
E = 320000
D = 128
R = 64
AGG = 32
OUT = 10

EB = 128
NUM_BATCHES = E // EB
NW = 32
BATCHES_PER_TILE = -(-NUM_BATCHES // NW)
NPAD = 10240
ROWS_PER_TILE = NPAD // 16
CHUNK = 128


PHASES = 2
PHASE_BATCHES = 40
IDX_ROWS = 3 * PHASES * PHASE_BATCHES
IDX_ROWS_VMEM = 3 * PHASE_BATCHES


def _sc_edge_accumulate(x, perm, w_conv):
    mesh = plsc.VectorSubcoreMesh(core_axis_name="c", subcore_axis_name="s")

    @functools.partial(
        pl.kernel,
        mesh=mesh,
        compiler_params=pltpu.CompilerParams(needs_layout_passes=False),
        out_type=jax.ShapeDtypeStruct((2, NPAD, D), jnp.float32),
        scratch_types=[
            pltpu.VMEM((IDX_ROWS_VMEM, EB), jnp.int32),
            pltpu.VMEM((EB, D), jnp.float32),
            pltpu.VMEM((EB, D), jnp.float32),
            pltpu.VMEM((128,), jnp.float32),
            pltpu.VMEM((128,), jnp.float32),
            pltpu.VMEM_SHARED((NPAD, D), jnp.float32),
            pltpu.SemaphoreType.DMA,
            pltpu.SemaphoreType.DMA,
        ],
    )
    def k(x_hbm, perm_hbm, wconv_hbm, out_hbm,
          idx_all, rows0, rows1, wconv_v, wg_v, h_sh, sem0, sem1):
        c = lax.axis_index("c")
        s = lax.axis_index("s")
        wid = s * 2 + c

        pltpu.sync_copy(wconv_hbm, wconv_v.at[pl.ds(0, R)])
        pltpu.sync_copy(perm_hbm.at[wid, pl.ds(0, IDX_ROWS_VMEM)], idx_all)

        z16 = jnp.zeros((16,), jnp.float32)

        def zrow(i, _):
            for jb in range(D // 16):
                rows0[i, pl.ds(jb * 16, 16)] = z16
            return 0

        lax.fori_loop(0, EB, zrow, 0)
        for t in range(ROWS_PER_TILE // CHUNK):
            pltpu.sync_copy(
                rows0.at[pl.ds(0, CHUNK)],
                h_sh.at[pl.ds(s * ROWS_PER_TILE + t * CHUNK, CHUNK)])
        plsc.subcore_barrier()

        def run_batch(kk, kbase, rows_v, sem, nrows_v, nsem):
            b = wid + (kbase + kk) * NW

            @pl.when(b < NUM_BATCHES)
            def _():
                r0 = 3 * kk
                pltpu.make_async_copy(
                    x_hbm.at[idx_all.at[r0]], rows_v, sem).wait()
                bn = b + NW

                @pl.when(jnp.logical_and(bn < NUM_BATCHES,
                                         kk < PHASE_BATCHES - 1))
                def _():
                    pltpu.async_copy(
                        x_hbm.at[idx_all.at[r0 + 3]], nrows_v, nsem)

                for g in range(EB // 16):
                    r16 = idx_all[r0 + 2, pl.ds(g * 16, 16)]
                    wg_v[pl.ds(g * 16, 16)] = plsc.load_gather(wconv_v, [r16])

                def scale_row(i, _):
                    i16 = jnp.zeros((16,), jnp.int32) + i
                    wb = plsc.load_gather(wg_v, [i16])
                    for jb in range(D // 16):
                        rows_v[i, pl.ds(jb * 16, 16)] = (
                            rows_v[i, pl.ds(jb * 16, 16)] * wb)
                    return 0

                lax.fori_loop(0, EB, scale_row, 0, unroll=8)
                pltpu.sync_copy(rows_v, h_sh.at[idx_all.at[r0 + 1]], add=True)

        for phase in range(PHASES):
            kbase = phase * PHASE_BATCHES
            pltpu.sync_copy(
                perm_hbm.at[wid, pl.ds(3 * kbase, 3 * PHASE_BATCHES)],
                idx_all)

            @pl.when(wid + kbase * NW < NUM_BATCHES)
            def _():
                pltpu.async_copy(x_hbm.at[idx_all.at[0]], rows0, sem0)

            def batch_pair(t, _, kbase=kbase):
                kk = 2 * t
                run_batch(kk, kbase, rows0, sem0, rows1, sem1)
                run_batch(kk + 1, kbase, rows1, sem1, rows0, sem0)
                return 0

            lax.fori_loop(0, PHASE_BATCHES // 2, batch_pair, 0)
        plsc.subcore_barrier()

        for t in range(ROWS_PER_TILE // CHUNK):
            r0 = s * ROWS_PER_TILE + t * CHUNK
            pltpu.sync_copy(h_sh.at[pl.ds(r0, CHUNK)],
                            rows0.at[pl.ds(0, CHUNK)])
            pltpu.sync_copy(rows0.at[pl.ds(0, CHUNK)],
                            out_hbm.at[c, pl.ds(r0, CHUNK)])

    return k(x, perm, w_conv)


NB = 1000
GRID = N // NB


def _tc_head(hp, nrA, nrB, bconv_row, w_agg, Wt, blin2):
    def body(hp_ref, nrA_ref, nrB_ref, bc_ref, wagg_ref, wt_ref, bl_ref,
             out_ref, s_ref):
        i = pl.program_id(0)

        @pl.when(i == 0)
        def _():
            s_ref[...] = jnp.zeros((AGG, D), jnp.float32)

        acc = hp_ref[0] + hp_ref[1]
        nrcol = nrA_ref[0]
        oh = (lax.broadcasted_iota(jnp.int32, (NB, R), 1) == nrcol
              ).astype(jnp.float32)
        bcol = jnp.sum(oh * bc_ref[...], axis=1, keepdims=True)
        h = jnp.tanh(acc + bcol)
        nrrow = nrB_ref[0]
        ohT = (lax.broadcasted_iota(jnp.int32, (R, NB), 0) == nrrow
               ).astype(jnp.float32)
        wn = jnp.dot(wagg_ref[...], ohT,
                     preferred_element_type=jnp.float32)
        s_ref[...] += jnp.dot(wn, h, preferred_element_type=jnp.float32)

        @pl.when(i == GRID - 1)
        def _():
            a = jnp.tanh(s_ref[...])
            acc10 = bl_ref[...]
            for k in range(AGG):
                acc10 = acc10 + jnp.dot(a[k:k + 1, :], wt_ref[k],
                                        preferred_element_type=jnp.float32)
            out_ref[...] = jnp.tanh(acc10)

    return pl.pallas_call(
        body,
        grid=(GRID,),
        in_specs=[
            pl.BlockSpec((2, NB, D), lambda i: (0, i, 0)),
            pl.BlockSpec((1, NB, 1), lambda i: (i, 0, 0)),
            pl.BlockSpec((1, 1, NB), lambda i: (i, 0, 0)),
            pl.BlockSpec((1, R), lambda i: (0, 0)),
            pl.BlockSpec((AGG, R), lambda i: (0, 0)),
            pl.BlockSpec((AGG, D, OUT), lambda i: (0, 0, 0)),
            pl.BlockSpec((1, OUT), lambda i: (0, 0)),
        ],
        out_specs=pl.BlockSpec((1, OUT), lambda i: (0, 0)),
        out_shape=jax.ShapeDtypeStruct((1, OUT), jnp.float32),
        scratch_shapes=[pltpu.VMEM((AGG, D), jnp.float32)],
    )(hp, nrA, nrB, bconv_row, w_agg, Wt, blin2)


def kernel(x, pos, edge_rule, node_rule, w_conv, b_conv, w_agg, W_lin, b_lin):
    src = pos[0].reshape(NUM_BATCHES, 1, EB)
    dst = pos[1].reshape(NUM_BATCHES, 1, EB)
    erule = edge_rule.reshape(NUM_BATCHES, 1, EB)
    packed = jnp.concatenate([src, dst, erule], axis=1)
    bidx = (jnp.arange(NW)[:, None] +
            jnp.arange(PHASES * PHASE_BATCHES)[None, :] * NW)
    bidx = jnp.where(bidx < NUM_BATCHES, bidx, 0)
    perm = packed[bidx].reshape(NW, IDX_ROWS, EB)
    hp = _sc_edge_accumulate(x, perm, w_conv)[:, :N, :]
    nrA = node_rule.reshape(GRID, NB, 1)
    nrB = node_rule.reshape(GRID, 1, NB)
    Wt = W_lin.reshape(OUT, AGG, D).transpose(1, 2, 0)
    out2 = _tc_head(hp, nrA, nrB, b_conv.reshape(1, R), w_agg, Wt,
                    b_lin.reshape(1, OUT))
    return out2.reshape(-1)
